# baseline (speedup 1.0000x reference)
import functools

import jax
import jax.numpy as jnp
from jax import lax
from jax.experimental import pallas as pl
from jax.experimental.pallas import tpu as pltpu
from jax.experimental.pallas import tpu_sc as plsc

NC = 2
NS = 16
NW = NC * NS


def _i16():
    return lax.iota(jnp.int32, 16)


@functools.lru_cache(maxsize=None)
def _make_phase_a(vocab, d):
    assert d == 32
    n_full = vocab // 128
    tail = vocab - n_full * 128
    per_worker = (n_full // NW) & ~3
    n_left = n_full - per_worker * NW
    assert n_left < NW

    mesh = plsc.VectorSubcoreMesh(core_axis_name="c", subcore_axis_name="s")

    @functools.partial(
        pl.kernel,
        mesh=mesh,
        out_type=jax.ShapeDtypeStruct((vocab * d,), jnp.float32),
        scratch_types=(
            [pltpu.VMEM((d, 128), jnp.float32)] * 4
            + [pltpu.VMEM((128 * d,), jnp.float32)] * 4
            + [pltpu.SemaphoreType.DMA] * 8
        ),
        compiler_params=pltpu.CompilerParams(needs_layout_passes=False),
    )
    def phase_a(wt, w_tail, w_lin, *bufs):
        chunks, lins = list(bufs[0:4]), list(bufs[4:8])
        isems, osems = list(bufs[8:12]), list(bufs[12:16])
        wid = lax.axis_index("s") * NC + lax.axis_index("c")
        c0 = wid * per_worker

        iot = _i16()
        jcol = [iot + jh * 16 for jh in (0, 1)]

        def transpose_col(chunk, lin):
            @plsc.parallel_loop(0, 16, unroll=2)
            def _(dg):
                bdiag = (iot + dg) & 15
                bdiag32 = bdiag * d
                for jh in range(2):
                    for b16 in range(8):
                        bvec = bdiag + b16 * 16
                        st = bdiag32 + jcol[jh] + b16 * 16 * d
                        vals = plsc.load_gather(chunk, [jcol[jh], bvec])
                        plsc.store_scatter(lin, [st], vals)

        def issue_in(s, k):
            return pltpu.async_copy(
                wt.at[:, pl.ds((c0 + s) * 128, 128)], chunks[k], isems[k]
            )

        for k in range(3):
            issue_in(k, k)

        def step(i, carry):
            for k in range(4):
                s = i * 4 + k
                c = c0 + s
                pltpu.make_async_copy(
                    wt.at[:, pl.ds(c * 128, 128)], chunks[k], isems[k]
                ).wait()

                @pl.when(s + 3 < per_worker)
                def _():
                    issue_in(s + 3, (k + 3) % 4)

                @pl.when(s >= 4)
                def _():
                    pltpu.make_async_copy(
                        lins[k], w_lin.at[pl.ds(0, 128 * d)], osems[k]
                    ).wait()

                transpose_col(chunks[k], lins[k])
                pltpu.async_copy(
                    lins[k], w_lin.at[pl.ds(c * 128 * d, 128 * d)], osems[k]
                )
            return carry

        lax.fori_loop(0, per_worker // 4, step, 0)
        for k in range(4):
            pltpu.make_async_copy(
                lins[k], w_lin.at[pl.ds(0, 128 * d)], osems[k]
            ).wait()

        @pl.when(wid < n_left)
        def _():
            c = n_full - n_left + wid
            pltpu.sync_copy(wt.at[:, pl.ds(c * 128, 128)], chunks[0])
            transpose_col(chunks[0], lins[0])
            pltpu.sync_copy(lins[0], w_lin.at[pl.ds(c * 128 * d, 128 * d)])

        if tail:
            @pl.when(wid == n_left)
            def _():
                pltpu.sync_copy(w_tail, lins[0].at[pl.ds(0, tail * d)])
                pltpu.sync_copy(
                    lins[0].at[pl.ds(0, tail * d)],
                    w_lin.at[pl.ds(n_full * 128 * d, tail * d)],
                )

    return phase_a


@functools.lru_cache(maxsize=None)
def _make_phase_b(n_ctx_t, n_q_t, vocab, d):
    assert d == 32
    bsz = 4096
    tc = n_ctx_t // bsz
    tq = n_q_t // bsz
    pw_lo = tc // NW
    n_hi = tc - pw_lo * NW
    assert n_hi < NW and tq <= NW

    mesh = plsc.VectorSubcoreMesh(core_axis_name="c", subcore_axis_name="s")

    @functools.partial(
        pl.kernel,
        mesh=mesh,
        out_type=[
            jax.ShapeDtypeStruct((tc, 4, 256, 128), jnp.float32),
            jax.ShapeDtypeStruct((tq, 4, 256, 128), jnp.float32),
        ],
        scratch_types=(
            [pltpu.VMEM(((pw_lo + 1) * bsz,), jnp.int32)]
            + [pltpu.VMEM((128, d), jnp.float32)] * 4
            + [pltpu.VMEM((512, 128), jnp.float32)]
            + [pltpu.SemaphoreType.DMA] * 5
        ),
        compiler_params=pltpu.CompilerParams(
            use_tc_tiling_on_sc=False, needs_layout_passes=False
        ),
    )
    def phase_b(table, idx_ctx, idx_q, out_c, out_q, idxv, r0, r1, r2, r3,
                plane, g0, g1, g2, g3, osem):
        rows, gsems = [r0, r1, r2, r3], [g0, g1, g2, g3]
        wid = lax.axis_index("s") * NC + lax.axis_index("c")
        extra = (wid < n_hi).astype(jnp.int32)
        pw = pw_lo + extra
        tstart = wid * pw_lo + jnp.minimum(wid, n_hi)

        iot = _i16()
        rowbase = [(iot // 8) * 128 + (iot % 8) + jh * 256 for jh in (0, 1)]
        jcol = [iot + jh * 16 for jh in (0, 1)]

        def transpose_block(rblk, bc):
            bc8 = jnp.full((16,), bc * 8, jnp.int32)
            rowv = [rowbase[0] + bc8, rowbase[1] + bc8]

            @plsc.parallel_loop(0, 16, unroll=2)
            def _(dg):
                bdiag = (iot + dg) & 15
                for jh in range(2):
                    for b16 in range(8):
                        bvec = bdiag + b16 * 16
                        vals = plsc.load_gather(rblk, [bvec, jcol[jh]])
                        plsc.store_scatter(plane, [rowv[jh], bvec], vals)

        def issue_g(base, bc, k):
            for h in range(2):
                pltpu.async_copy(
                    table.at[idxv.at[pl.ds(base + bc * 128 + h * 64, 64)]],
                    rows[k].at[pl.ds(h * 64, 64)], gsems[k],
                )

        def half_body(out5, t, bh, local_half, first):
            @pl.when(jnp.logical_not(first))
            def _():
                for jt in range(4):
                    pltpu.make_async_copy(
                        plane.at[pl.ds(jt * 128, 128)],
                        out5.at[0, jt, pl.ds(0, 128)], osem,
                    ).wait()

            base = local_half * 2048
            for k in range(3):
                issue_g(base, k, k)

            def bc_step(i, carry):
                for k in range(4):
                    bc = i * 4 + k
                    for h in range(2):
                        pltpu.make_async_copy(
                            table.at[idxv.at[pl.ds(base, 64)]],
                            rows[k].at[pl.ds(h * 64, 64)], gsems[k],
                        ).wait()

                    @pl.when(bc + 3 < 16)
                    def _():
                        issue_g(base, bc + 3, (k + 3) % 4)

                    transpose_block(rows[k], bc)
                return carry

            lax.fori_loop(0, 4, bc_step, 0)
            for jt in range(4):
                pltpu.async_copy(
                    plane.at[pl.ds(jt * 128, 128)],
                    out5.at[t, jt, pl.ds(bh * 128, 128)], osem,
                )

        def drain_out(out5):
            for jt in range(4):
                pltpu.make_async_copy(
                    plane.at[pl.ds(jt * 128, 128)],
                    out5.at[0, jt, pl.ds(0, 128)], osem,
                ).wait()

        pltpu.sync_copy(
            idx_ctx.at[pl.ds(tstart * bsz, pw_lo * bsz)],
            idxv.at[pl.ds(0, pw_lo * bsz)],
        )

        @pl.when(extra == 1)
        def _():
            pltpu.sync_copy(
                idx_ctx.at[pl.ds((tstart + pw_lo) * bsz, bsz)],
                idxv.at[pl.ds(pw_lo * bsz, bsz)],
            )

        def ctx_half(h, carry):
            @pl.when(h < pw * 2)
            def _():
                half_body(out_c, tstart + h // 2, h % 2, h, h == 0)
            return carry

        lax.fori_loop(0, (pw_lo + 1) * 2, ctx_half, 0)
        drain_out(out_c)

        @pl.when(wid < tq)
        def _():
            pltpu.sync_copy(
                idx_q.at[pl.ds(wid * bsz, bsz)], idxv.at[pl.ds(0, bsz)]
            )

            def q_half(h, carry):
                half_body(out_q, wid, h, h, h == 0)
                return carry

            lax.fori_loop(0, 2, q_half, 0)
            drain_out(out_q)

    return phase_b


def kernel(input_context, input_question, word_embedding_weight):
    batch, ctx_len = input_context.shape
    _, q_len = input_question.shape
    vocab, d = word_embedding_weight.shape

    wt = word_embedding_weight.T
    n_full = vocab // 128
    w_tail = word_embedding_weight[n_full * 128:].reshape(-1)
    w_lin = _make_phase_a(vocab, d)(wt, w_tail)
    table = w_lin.reshape(vocab, d)

    idx_ctx = input_context.astype(jnp.int32).T.reshape(-1)
    idx_q = input_question.astype(jnp.int32).T.reshape(-1)

    out4c, out4q = _make_phase_b(
        batch * ctx_len, batch * q_len, vocab, d
    )(table, idx_ctx, idx_q)

    out5c = out4c.reshape(ctx_len, 4, 32, 8, 128)
    out5q = out4q.reshape(q_len, 4, 32, 8, 128)
    octx = out5c.transpose(2, 4, 0, 1, 3).reshape(batch, ctx_len, d)
    oq = out5q.transpose(2, 4, 0, 1, 3).reshape(batch, q_len, d)
    return (octx, oq)


# speedup vs baseline: 1.0823x; 1.0823x over previous
import functools

import jax
import jax.numpy as jnp
from jax import lax
from jax.experimental import pallas as pl
from jax.experimental.pallas import tpu as pltpu
from jax.experimental.pallas import tpu_sc as plsc

NC = 2
NS = 16
NW = NC * NS


def _i16():
    return lax.iota(jnp.int32, 16)


@functools.lru_cache(maxsize=None)
def _make_phase_a(vocab, d):
    assert d == 32
    n_full = vocab // 128
    tail = vocab - n_full * 128
    per_worker = (n_full // NW) & ~3
    n_left = n_full - per_worker * NW
    assert n_left < NW

    mesh = plsc.VectorSubcoreMesh(core_axis_name="c", subcore_axis_name="s")

    @functools.partial(
        pl.kernel,
        mesh=mesh,
        out_type=jax.ShapeDtypeStruct((vocab * d,), jnp.float32),
        scratch_types=(
            [pltpu.VMEM((d, 128), jnp.float32)] * 4
            + [pltpu.VMEM((128 * d,), jnp.float32)] * 4
            + [pltpu.SemaphoreType.DMA] * 8
        ),
        compiler_params=pltpu.CompilerParams(needs_layout_passes=False),
    )
    def phase_a(wt, w_tail, w_lin, *bufs):
        chunks, lins = list(bufs[0:4]), list(bufs[4:8])
        isems, osems = list(bufs[8:12]), list(bufs[12:16])
        wid = lax.axis_index("s") * NC + lax.axis_index("c")
        c0 = wid * per_worker

        iot = _i16()
        jcol = [iot + jh * 16 for jh in (0, 1)]

        def transpose_col(chunk, lin):
            @plsc.parallel_loop(0, 16, unroll=2)
            def _(dg):
                bdiag = (iot + dg) & 15
                bdiag32 = bdiag * d
                for jh in range(2):
                    for b16 in range(8):
                        bvec = bdiag + b16 * 16
                        st = bdiag32 + jcol[jh] + b16 * 16 * d
                        vals = plsc.load_gather(chunk, [jcol[jh], bvec])
                        plsc.store_scatter(lin, [st], vals)

        def issue_in(s, k):
            return pltpu.async_copy(
                wt.at[:, pl.ds((c0 + s) * 128, 128)], chunks[k], isems[k]
            )

        for k in range(3):
            issue_in(k, k)

        def step(i, carry):
            for k in range(4):
                s = i * 4 + k
                c = c0 + s
                pltpu.make_async_copy(
                    wt.at[:, pl.ds(c * 128, 128)], chunks[k], isems[k]
                ).wait()

                @pl.when(s + 3 < per_worker)
                def _():
                    issue_in(s + 3, (k + 3) % 4)

                @pl.when(s >= 4)
                def _():
                    pltpu.make_async_copy(
                        lins[k], w_lin.at[pl.ds(0, 128 * d)], osems[k]
                    ).wait()

                transpose_col(chunks[k], lins[k])
                pltpu.async_copy(
                    lins[k], w_lin.at[pl.ds(c * 128 * d, 128 * d)], osems[k]
                )
            return carry

        lax.fori_loop(0, per_worker // 4, step, 0)
        for k in range(4):
            pltpu.make_async_copy(
                lins[k], w_lin.at[pl.ds(0, 128 * d)], osems[k]
            ).wait()

        @pl.when(wid < n_left)
        def _():
            c = n_full - n_left + wid
            pltpu.sync_copy(wt.at[:, pl.ds(c * 128, 128)], chunks[0])
            transpose_col(chunks[0], lins[0])
            pltpu.sync_copy(lins[0], w_lin.at[pl.ds(c * 128 * d, 128 * d)])

        if tail:
            @pl.when(wid == n_left)
            def _():
                pltpu.sync_copy(w_tail, lins[0].at[pl.ds(0, tail * d)])
                pltpu.sync_copy(
                    lins[0].at[pl.ds(0, tail * d)],
                    w_lin.at[pl.ds(n_full * 128 * d, tail * d)],
                )

    return phase_a


@functools.lru_cache(maxsize=None)
def _make_phase_b(n_ctx_t, n_q_t, vocab, d):
    assert d == 32
    bsz = 4096
    tc = n_ctx_t // bsz
    tq = n_q_t // bsz
    pw_lo = tc // NW
    n_hi = tc - pw_lo * NW
    assert n_hi < NW and tq <= NW

    mesh = plsc.VectorSubcoreMesh(core_axis_name="c", subcore_axis_name="s")

    @functools.partial(
        pl.kernel,
        mesh=mesh,
        out_type=[
            jax.ShapeDtypeStruct((tc, 4, 256, 128), jnp.float32),
            jax.ShapeDtypeStruct((tq, 4, 256, 128), jnp.float32),
        ],
        scratch_types=(
            [pltpu.VMEM(((pw_lo + 1) * bsz,), jnp.int32)]
            + [pltpu.VMEM((128, d), jnp.float32)] * 4
            + [pltpu.VMEM((512, 128), jnp.float32)]
            + [pltpu.SemaphoreType.DMA] * 5
        ),
        compiler_params=pltpu.CompilerParams(
            use_tc_tiling_on_sc=False, needs_layout_passes=False
        ),
    )
    def phase_b(table, idx_ctx, idx_q, out_c, out_q, idxv, r0, r1, r2, r3,
                plane, g0, g1, g2, g3, osem):
        rows, gsems = [r0, r1, r2, r3], [g0, g1, g2, g3]
        wid = lax.axis_index("s") * NC + lax.axis_index("c")
        extra = (wid < n_hi).astype(jnp.int32)
        pw = pw_lo + extra
        tstart = wid * pw_lo + jnp.minimum(wid, n_hi)

        iot = _i16()
        rowbase = [(iot // 8) * 128 + (iot % 8) + jh * 256 for jh in (0, 1)]
        jcol = [iot + jh * 16 for jh in (0, 1)]

        def transpose_block(rblk, bc):
            bc8 = jnp.full((16,), bc * 8, jnp.int32)
            rowv = [rowbase[0] + bc8, rowbase[1] + bc8]

            @plsc.parallel_loop(0, 16, unroll=4)
            def _(dg):
                bdiag = (iot + dg) & 15
                for jh in range(2):
                    for b16 in range(8):
                        bvec = bdiag + b16 * 16
                        vals = plsc.load_gather(rblk, [bvec, jcol[jh]])
                        plsc.store_scatter(plane, [rowv[jh], bvec], vals)

        def issue_g(base, bc, k):
            for h in range(2):
                pltpu.async_copy(
                    table.at[idxv.at[pl.ds(base + bc * 128 + h * 64, 64)]],
                    rows[k].at[pl.ds(h * 64, 64)], gsems[k],
                )

        def half_body(out5, t, bh, local_half, first):
            @pl.when(jnp.logical_not(first))
            def _():
                for jt in range(4):
                    pltpu.make_async_copy(
                        plane.at[pl.ds(jt * 128, 128)],
                        out5.at[0, jt, pl.ds(0, 128)], osem,
                    ).wait()

            base = local_half * 2048
            for k in range(3):
                issue_g(base, k, k)

            def bc_step(i, carry):
                for k in range(4):
                    bc = i * 4 + k
                    for h in range(2):
                        pltpu.make_async_copy(
                            table.at[idxv.at[pl.ds(base, 64)]],
                            rows[k].at[pl.ds(h * 64, 64)], gsems[k],
                        ).wait()

                    @pl.when(bc + 3 < 16)
                    def _():
                        issue_g(base, bc + 3, (k + 3) % 4)

                    transpose_block(rows[k], bc)
                return carry

            lax.fori_loop(0, 4, bc_step, 0)
            for jt in range(4):
                pltpu.async_copy(
                    plane.at[pl.ds(jt * 128, 128)],
                    out5.at[t, jt, pl.ds(bh * 128, 128)], osem,
                )

        def drain_out(out5):
            for jt in range(4):
                pltpu.make_async_copy(
                    plane.at[pl.ds(jt * 128, 128)],
                    out5.at[0, jt, pl.ds(0, 128)], osem,
                ).wait()

        pltpu.sync_copy(
            idx_ctx.at[pl.ds(tstart * bsz, pw_lo * bsz)],
            idxv.at[pl.ds(0, pw_lo * bsz)],
        )

        @pl.when(extra == 1)
        def _():
            pltpu.sync_copy(
                idx_ctx.at[pl.ds((tstart + pw_lo) * bsz, bsz)],
                idxv.at[pl.ds(pw_lo * bsz, bsz)],
            )

        def ctx_half(h, carry):
            @pl.when(h < pw * 2)
            def _():
                half_body(out_c, tstart + h // 2, h % 2, h, h == 0)
            return carry

        lax.fori_loop(0, (pw_lo + 1) * 2, ctx_half, 0)
        drain_out(out_c)

        @pl.when(wid < tq)
        def _():
            pltpu.sync_copy(
                idx_q.at[pl.ds(wid * bsz, bsz)], idxv.at[pl.ds(0, bsz)]
            )

            def q_half(h, carry):
                half_body(out_q, wid, h, h, h == 0)
                return carry

            lax.fori_loop(0, 2, q_half, 0)
            drain_out(out_q)

    return phase_b


def kernel(input_context, input_question, word_embedding_weight):
    batch, ctx_len = input_context.shape
    _, q_len = input_question.shape
    vocab, d = word_embedding_weight.shape

    wt = word_embedding_weight.T
    n_full = vocab // 128
    w_tail = word_embedding_weight[n_full * 128:].reshape(-1)
    w_lin = _make_phase_a(vocab, d)(wt, w_tail)
    table = w_lin.reshape(vocab, d)

    idx_ctx = input_context.astype(jnp.int32).T.reshape(-1)
    idx_q = input_question.astype(jnp.int32).T.reshape(-1)

    out4c, out4q = _make_phase_b(
        batch * ctx_len, batch * q_len, vocab, d
    )(table, idx_ctx, idx_q)

    out5c = out4c.reshape(ctx_len, 4, 32, 8, 128)
    out5q = out4q.reshape(q_len, 4, 32, 8, 128)
    octx = out5c.transpose(2, 4, 0, 1, 3).reshape(batch, ctx_len, d)
    oq = out5q.transpose(2, 4, 0, 1, 3).reshape(batch, q_len, d)
    return (octx, oq)


# speedup vs baseline: 1.3729x; 1.2686x over previous
import functools

import jax
import jax.numpy as jnp
from jax import lax
from jax.experimental import pallas as pl
from jax.experimental.pallas import tpu as pltpu
from jax.experimental.pallas import tpu_sc as plsc

NC = 2
NS = 16
NW = NC * NS


def _i16():
    return lax.iota(jnp.int32, 16)


@functools.lru_cache(maxsize=None)
def _make_phase_a(vocab, d):
    assert d == 32
    n_full = vocab // 128
    tail = vocab - n_full * 128
    per_worker = (n_full // NW) & ~3
    n_left = n_full - per_worker * NW
    assert n_left < NW

    mesh = plsc.VectorSubcoreMesh(core_axis_name="c", subcore_axis_name="s")

    @functools.partial(
        pl.kernel,
        mesh=mesh,
        out_type=jax.ShapeDtypeStruct((vocab * d,), jnp.float32),
        scratch_types=(
            [pltpu.VMEM((d, 128), jnp.float32)] * 4
            + [pltpu.VMEM((128 * d,), jnp.float32)] * 4
            + [pltpu.SemaphoreType.DMA] * 8
        ),
        compiler_params=pltpu.CompilerParams(needs_layout_passes=False),
    )
    def phase_a(wt, w_tail, w_lin, *bufs):
        chunks, lins = list(bufs[0:4]), list(bufs[4:8])
        isems, osems = list(bufs[8:12]), list(bufs[12:16])
        wid = lax.axis_index("s") * NC + lax.axis_index("c")
        c0 = wid * per_worker

        iot = _i16()
        jcol = [iot + jh * 16 for jh in (0, 1)]

        def transpose_col(chunk, lin):
            @plsc.parallel_loop(0, 16, unroll=4)
            def _(dg):
                bdiag = (iot + dg) & 15
                bdiag32 = bdiag * d
                for jh in range(2):
                    for b16 in range(8):
                        bvec = bdiag + b16 * 16
                        st = bdiag32 + jcol[jh] + b16 * 16 * d
                        vals = plsc.load_gather(chunk, [jcol[jh], bvec])
                        plsc.store_scatter(lin, [st], vals)

        def issue_in(s, k):
            return pltpu.async_copy(
                wt.at[:, pl.ds((c0 + s) * 128, 128)], chunks[k], isems[k]
            )

        for k in range(3):
            issue_in(k, k)

        def step(i, carry):
            for k in range(4):
                s = i * 4 + k
                c = c0 + s
                pltpu.make_async_copy(
                    wt.at[:, pl.ds(c * 128, 128)], chunks[k], isems[k]
                ).wait()

                @pl.when(s + 3 < per_worker)
                def _():
                    issue_in(s + 3, (k + 3) % 4)

                @pl.when(s >= 4)
                def _():
                    pltpu.make_async_copy(
                        lins[k], w_lin.at[pl.ds(0, 128 * d)], osems[k]
                    ).wait()

                transpose_col(chunks[k], lins[k])
                pltpu.async_copy(
                    lins[k], w_lin.at[pl.ds(c * 128 * d, 128 * d)], osems[k]
                )
            return carry

        lax.fori_loop(0, per_worker // 4, step, 0)
        for k in range(4):
            pltpu.make_async_copy(
                lins[k], w_lin.at[pl.ds(0, 128 * d)], osems[k]
            ).wait()

        @pl.when(wid < n_left)
        def _():
            c = n_full - n_left + wid
            pltpu.sync_copy(wt.at[:, pl.ds(c * 128, 128)], chunks[0])
            transpose_col(chunks[0], lins[0])
            pltpu.sync_copy(lins[0], w_lin.at[pl.ds(c * 128 * d, 128 * d)])

        if tail:
            @pl.when(wid == n_left)
            def _():
                pltpu.sync_copy(w_tail, lins[0].at[pl.ds(0, tail * d)])
                pltpu.sync_copy(
                    lins[0].at[pl.ds(0, tail * d)],
                    w_lin.at[pl.ds(n_full * 128 * d, tail * d)],
                )

    return phase_a


@functools.lru_cache(maxsize=None)
def _make_phase_b(n_ctx_t, n_q_t, vocab, d):
    assert d == 32
    bsz = 4096
    tc = n_ctx_t // bsz
    tq = n_q_t // bsz
    pw_lo = tc // NW
    n_hi = tc - pw_lo * NW
    assert n_hi < NW and tq <= NW

    mesh = plsc.VectorSubcoreMesh(core_axis_name="c", subcore_axis_name="s")

    @functools.partial(
        pl.kernel,
        mesh=mesh,
        out_type=[
            jax.ShapeDtypeStruct((tc, 4, 256, 128), jnp.float32),
            jax.ShapeDtypeStruct((tq, 4, 256, 128), jnp.float32),
        ],
        scratch_types=(
            [pltpu.VMEM(((pw_lo + 1) * bsz,), jnp.int32)]
            + [pltpu.VMEM((128, d), jnp.float32)] * 4
            + [pltpu.VMEM((512, 128), jnp.float32)]
            + [pltpu.SemaphoreType.DMA] * 5
        ),
        compiler_params=pltpu.CompilerParams(
            use_tc_tiling_on_sc=False, needs_layout_passes=False
        ),
    )
    def phase_b(table, idx_ctx, idx_q, out_c, out_q, idxv, r0, r1, r2, r3,
                plane, g0, g1, g2, g3, osem):
        rows, gsems = [r0, r1, r2, r3], [g0, g1, g2, g3]
        wid = lax.axis_index("s") * NC + lax.axis_index("c")
        extra = (wid < n_hi).astype(jnp.int32)
        pw = pw_lo + extra
        tstart = wid * pw_lo + jnp.minimum(wid, n_hi)

        iot = _i16()
        rowbase = [(iot // 8) * 128 + (iot % 8) + jh * 256 for jh in (0, 1)]
        jcol = [iot + jh * 16 for jh in (0, 1)]

        def transpose_block(rblk, bc):
            bc8 = jnp.full((16,), bc * 8, jnp.int32)
            rowv = [rowbase[0] + bc8, rowbase[1] + bc8]

            @plsc.parallel_loop(0, 16, unroll=8)
            def _(dg):
                bdiag = (iot + dg) & 15
                for jh in range(2):
                    for b16 in range(8):
                        bvec = bdiag + b16 * 16
                        vals = plsc.load_gather(rblk, [bvec, jcol[jh]])
                        plsc.store_scatter(plane, [rowv[jh], bvec], vals)

        def issue_g(base, bc, k):
            for h in range(2):
                pltpu.async_copy(
                    table.at[idxv.at[pl.ds(base + bc * 128 + h * 64, 64)]],
                    rows[k].at[pl.ds(h * 64, 64)], gsems[k],
                )

        def half_body(out5, t, bh, local_half, first):
            @pl.when(jnp.logical_not(first))
            def _():
                for jt in range(4):
                    pltpu.make_async_copy(
                        plane.at[pl.ds(jt * 128, 128)],
                        out5.at[0, jt, pl.ds(0, 128)], osem,
                    ).wait()

            base = local_half * 2048
            for k in range(3):
                issue_g(base, k, k)

            def bc_step(i, carry):
                for k in range(4):
                    bc = i * 4 + k
                    for h in range(2):
                        pltpu.make_async_copy(
                            table.at[idxv.at[pl.ds(base, 64)]],
                            rows[k].at[pl.ds(h * 64, 64)], gsems[k],
                        ).wait()

                    @pl.when(bc + 3 < 16)
                    def _():
                        issue_g(base, bc + 3, (k + 3) % 4)

                    transpose_block(rows[k], bc)
                return carry

            lax.fori_loop(0, 4, bc_step, 0)
            for jt in range(4):
                pltpu.async_copy(
                    plane.at[pl.ds(jt * 128, 128)],
                    out5.at[t, jt, pl.ds(bh * 128, 128)], osem,
                )

        def drain_out(out5):
            for jt in range(4):
                pltpu.make_async_copy(
                    plane.at[pl.ds(jt * 128, 128)],
                    out5.at[0, jt, pl.ds(0, 128)], osem,
                ).wait()

        pltpu.sync_copy(
            idx_ctx.at[pl.ds(tstart * bsz, pw_lo * bsz)],
            idxv.at[pl.ds(0, pw_lo * bsz)],
        )

        @pl.when(extra == 1)
        def _():
            pltpu.sync_copy(
                idx_ctx.at[pl.ds((tstart + pw_lo) * bsz, bsz)],
                idxv.at[pl.ds(pw_lo * bsz, bsz)],
            )

        def ctx_half(h, carry):
            @pl.when(h < pw * 2)
            def _():
                half_body(out_c, tstart + h // 2, h % 2, h, h == 0)
            return carry

        lax.fori_loop(0, (pw_lo + 1) * 2, ctx_half, 0)
        drain_out(out_c)

        @pl.when(wid < tq)
        def _():
            pltpu.sync_copy(
                idx_q.at[pl.ds(wid * bsz, bsz)], idxv.at[pl.ds(0, bsz)]
            )

            def q_half(h, carry):
                half_body(out_q, wid, h, h, h == 0)
                return carry

            lax.fori_loop(0, 2, q_half, 0)
            drain_out(out_q)

    return phase_b


def kernel(input_context, input_question, word_embedding_weight):
    batch, ctx_len = input_context.shape
    _, q_len = input_question.shape
    vocab, d = word_embedding_weight.shape

    wt = word_embedding_weight.T
    n_full = vocab // 128
    w_tail = word_embedding_weight[n_full * 128:].reshape(-1)
    w_lin = _make_phase_a(vocab, d)(wt, w_tail)
    table = w_lin.reshape(vocab, d)

    idx_ctx = input_context.astype(jnp.int32).T.reshape(-1)
    idx_q = input_question.astype(jnp.int32).T.reshape(-1)

    out4c, out4q = _make_phase_b(
        batch * ctx_len, batch * q_len, vocab, d
    )(table, idx_ctx, idx_q)

    out5c = out4c.reshape(ctx_len, 4, 32, 8, 128)
    out5q = out4q.reshape(q_len, 4, 32, 8, 128)
    octx = out5c.transpose(2, 4, 0, 1, 3).reshape(batch, ctx_len, d)
    oq = out5q.transpose(2, 4, 0, 1, 3).reshape(batch, q_len, d)
    return (octx, oq)
